# baseline (speedup 1.0000x reference)
import functools

import jax
import jax.numpy as jnp
from jax.experimental import pallas as pl

_UNROLLED_ITERS = 6
_MAX_EXTRA_ITERS = 26


def _sparsemax_block(x_ref, o_ref):
    x = x_ref[...]

    def newton(tau):
        mask = x > tau
        g = jnp.where(mask, x - tau, 0.0)
        s = jnp.sum(g, axis=-1, keepdims=True)
        c = jnp.sum(mask.astype(jnp.float32), axis=-1, keepdims=True)
        return tau + (s - 1.0) / c

    tau = jnp.max(x, axis=-1, keepdims=True) - 1.0
    for _ in range(_UNROLLED_ITERS):
        tau = newton(tau)

    def cond(carry):
        k, _, changed = carry
        return jnp.logical_and(k < _MAX_EXTRA_ITERS, changed)

    def body(carry):
        k, tau, _ = carry
        tau_new = newton(tau)
        return k + 1, tau_new, jnp.any(tau_new != tau)

    _, tau, _ = jax.lax.while_loop(cond, body, (0, tau, jnp.bool_(True)))
    o_ref[...] = jnp.maximum(x - tau, 0.0)


_ROW_BLOCK = 8


@functools.partial(jax.jit, static_argnames=())
def kernel(x):
    rows, cols = x.shape
    return pl.pallas_call(
        _sparsemax_block,
        grid=(rows // _ROW_BLOCK,),
        in_specs=[pl.BlockSpec((_ROW_BLOCK, cols), lambda i: (i, 0))],
        out_specs=pl.BlockSpec((_ROW_BLOCK, cols), lambda i: (i, 0)),
        out_shape=jax.ShapeDtypeStruct(x.shape, x.dtype),
    )(x)
